# baseline (speedup 1.0000x reference)
import jax
import jax.numpy as jnp
from jax.experimental import pallas as pl
from jax.experimental.pallas import tpu as pltpu

BLOCK_ROWS = 512


def _body(ctx1_ref, ctx2_ref, w11_ref, b11_ref, w12_ref, b12_ref,
          w21_ref, b21_ref, w22_ref, b22_ref, x_ref,
          o1_ref, o2_ref, g1_scr, g2_scr):
    i = pl.program_id(0)

    @pl.when(i == 0)
    def _():
        h1 = jnp.maximum(
            jnp.dot(ctx1_ref[...], w11_ref[...],
                    preferred_element_type=jnp.float32) + b11_ref[...], 0.0)
        g1 = jax.nn.sigmoid(
            jnp.dot(h1, w12_ref[...],
                    preferred_element_type=jnp.float32) + b12_ref[...]) * 2.0
        g1_scr[...] = g1
        h2 = jnp.maximum(
            jnp.dot(ctx2_ref[...], w21_ref[...],
                    preferred_element_type=jnp.float32) + b21_ref[...], 0.0)
        g2 = jax.nn.sigmoid(
            jnp.dot(h2, w22_ref[...],
                    preferred_element_type=jnp.float32) + b22_ref[...]) * 2.0
        g2_scr[...] = g2

    x = x_ref[...]
    o1_ref[...] = x * g1_scr[...]
    o2_ref[...] = x * g2_scr[...]


def kernel(feed_dict, flat_emb, fs1_ctx_bias, fs2_ctx_bias,
           fs1_W1, fs1_b1, fs1_W2, fs1_b2,
           fs2_W1, fs2_b1, fs2_W2, fs2_b2):
    B, L, D = flat_emb.shape
    E = fs1_ctx_bias.shape[-1]
    H = fs1_W1.shape[-1]
    n = B * L
    x = flat_emb.reshape(n, D)

    grid = (n // BLOCK_ROWS,)
    const_spec = lambda shape: pl.BlockSpec(shape, lambda i: (0, 0))

    out1, out2 = pl.pallas_call(
        _body,
        grid=grid,
        in_specs=[
            const_spec((1, E)),
            const_spec((1, E)),
            const_spec((E, H)),
            const_spec((1, H)),
            const_spec((H, D)),
            const_spec((1, D)),
            const_spec((E, H)),
            const_spec((1, H)),
            const_spec((H, D)),
            const_spec((1, D)),
            pl.BlockSpec((BLOCK_ROWS, D), lambda i: (i, 0)),
        ],
        out_specs=[
            pl.BlockSpec((BLOCK_ROWS, D), lambda i: (i, 0)),
            pl.BlockSpec((BLOCK_ROWS, D), lambda i: (i, 0)),
        ],
        out_shape=[
            jax.ShapeDtypeStruct((n, D), jnp.float32),
            jax.ShapeDtypeStruct((n, D), jnp.float32),
        ],
        scratch_shapes=[
            pltpu.VMEM((1, D), jnp.float32),
            pltpu.VMEM((1, D), jnp.float32),
        ],
    )(fs1_ctx_bias, fs2_ctx_bias,
      fs1_W1, fs1_b1.reshape(1, H), fs1_W2, fs1_b2.reshape(1, D),
      fs2_W1, fs2_b1.reshape(1, H), fs2_W2, fs2_b2.reshape(1, D),
      x)

    return (out1.reshape(B, L, D), out2.reshape(B, L, D))


# speedup vs baseline: 1.6623x; 1.6623x over previous
import jax
import jax.numpy as jnp
from jax.experimental import pallas as pl
from jax.experimental.pallas import tpu as pltpu

BLOCK_B = 32


def _body(ctx1_ref, ctx2_ref, w11_ref, b11_ref, w12_ref, b12_ref,
          w21_ref, b21_ref, w22_ref, b22_ref, x_ref,
          o1_ref, o2_ref, g1_scr, g2_scr):
    i = pl.program_id(0)

    @pl.when(i == 0)
    def _():
        h1 = jnp.maximum(
            jnp.dot(ctx1_ref[...], w11_ref[...],
                    preferred_element_type=jnp.float32) + b11_ref[...], 0.0)
        g1 = jax.nn.sigmoid(
            jnp.dot(h1, w12_ref[...],
                    preferred_element_type=jnp.float32) + b12_ref[...]) * 2.0
        g1_scr[...] = g1
        h2 = jnp.maximum(
            jnp.dot(ctx2_ref[...], w21_ref[...],
                    preferred_element_type=jnp.float32) + b21_ref[...], 0.0)
        g2 = jax.nn.sigmoid(
            jnp.dot(h2, w22_ref[...],
                    preferred_element_type=jnp.float32) + b22_ref[...]) * 2.0
        g2_scr[...] = g2

    x = x_ref[...]
    g1 = g1_scr[...][None]
    g2 = g2_scr[...][None]
    o1_ref[...] = x * g1
    o2_ref[...] = x * g2


def kernel(feed_dict, flat_emb, fs1_ctx_bias, fs2_ctx_bias,
           fs1_W1, fs1_b1, fs1_W2, fs1_b2,
           fs2_W1, fs2_b1, fs2_W2, fs2_b2):
    B, L, D = flat_emb.shape
    E = fs1_ctx_bias.shape[-1]
    H = fs1_W1.shape[-1]

    grid = (B // BLOCK_B,)
    const_spec = lambda shape: pl.BlockSpec(shape, lambda i: (0, 0))

    out1, out2 = pl.pallas_call(
        _body,
        grid=grid,
        in_specs=[
            const_spec((1, E)),
            const_spec((1, E)),
            const_spec((E, H)),
            const_spec((1, H)),
            const_spec((H, D)),
            const_spec((1, D)),
            const_spec((E, H)),
            const_spec((1, H)),
            const_spec((H, D)),
            const_spec((1, D)),
            pl.BlockSpec((BLOCK_B, L, D), lambda i: (i, 0, 0)),
        ],
        out_specs=[
            pl.BlockSpec((BLOCK_B, L, D), lambda i: (i, 0, 0)),
            pl.BlockSpec((BLOCK_B, L, D), lambda i: (i, 0, 0)),
        ],
        out_shape=[
            jax.ShapeDtypeStruct((B, L, D), jnp.float32),
            jax.ShapeDtypeStruct((B, L, D), jnp.float32),
        ],
        scratch_shapes=[
            pltpu.VMEM((1, D), jnp.float32),
            pltpu.VMEM((1, D), jnp.float32),
        ],
    )(fs1_ctx_bias, fs2_ctx_bias,
      fs1_W1, fs1_b1.reshape(1, H), fs1_W2, fs1_b2.reshape(1, D),
      fs2_W1, fs2_b1.reshape(1, H), fs2_W2, fs2_b2.reshape(1, D),
      flat_emb)

    return (out1, out2)
